# baseline (speedup 1.0000x reference)
import functools

import jax
import jax.numpy as jnp
from jax import lax
from jax.experimental import pallas as pl
from jax.experimental.pallas import tpu as pltpu
from jax.experimental.pallas import tpu_sc as plsc

T, H, IDIM, E, TOPK = 2048, 1024, 2048, 8, 2

NC, NS, L = 2, 16, 16
NW = NC * NS
TPW = T // NW
NG = TPW // L

B = 128
PAIRS_C = T * TOPK // NC
NBLK_C = PAIRS_C // B + E
SREG = NBLK_C * B
NR = NC * SREG
NBLK = NC * NBLK_C

_mesh = plsc.VectorSubcoreMesh(core_axis_name="c", subcore_axis_name="s")


def _zi(v):
    return jnp.zeros((L,), jnp.int32) + v


def _zf(v):
    return jnp.zeros((L,), jnp.float32) + v


_GDN = lax.GatherDimensionNumbers(offset_dims=(), collapsed_slice_dims=(0,),
                                  start_index_map=(0,))


def _dyn_gather(vec, idx):
    return lax.gather(vec, idx[:, None], _GDN, (1,),
                      mode=lax.GatherScatterMode.PROMISE_IN_BOUNDS)


def _routing_body(hs, lg, xs, s0, s1, w0, w1, cnt,
                  lg_v, id0_v, id1_v, s0_v, s1_v, w0_v, w1_v,
                  cntv_v, allcnt_v, xbuf_v, shared_cnt, dsem):
    c = lax.axis_index("c")
    s = lax.axis_index("s")
    tbase = c * (T // NC) + s * TPW
    iota16 = lax.iota(jnp.int32, L)

    pltpu.sync_copy(lg.at[pl.ds(tbase * E, TPW * E)], lg_v)
    cnt_sc = [jnp.int32(0)] * E
    for g in range(NG):
        lvec = [plsc.load_gather(lg_v, [iota16 * E + (g * L * E + e)])
                for e in range(E)]
        a1 = lvec[0]
        i1 = _zi(0)
        a2 = _zf(-jnp.inf)
        i2 = _zi(0)
        for e in range(1, E):
            le = lvec[e]
            gt1 = le > a1
            gt2 = le > a2
            ef = _zi(e)
            i2 = jnp.where(gt1, i1, jnp.where(gt2, ef, i2))
            a2 = jnp.where(gt1, a1, jnp.where(gt2, le, a2))
            i1 = jnp.where(gt1, ef, i1)
            a1 = jnp.where(gt1, le, a1)
        p2 = jnp.exp(a2 - a1)
        wt1 = 1.0 / (1.0 + p2)
        id0_v[pl.ds(g * L, L)] = i1
        id1_v[pl.ds(g * L, L)] = i2
        w0_v[pl.ds(g * L, L)] = wt1
        w1_v[pl.ds(g * L, L)] = p2 * wt1
        for e in range(E):
            m = (i1 == e).astype(jnp.int32) + (i2 == e).astype(jnp.int32)
            cnt_sc[e] = cnt_sc[e] + jnp.sum(m)

    cnt_vec = _zi(0)
    for e in range(E):
        cnt_vec = jnp.where(iota16 == e, _zi(cnt_sc[e]), cnt_vec)
    cntv_v[...] = cnt_vec
    pltpu.sync_copy(cntv_v, shared_cnt.at[pl.ds(s * L, L)])
    plsc.subcore_barrier()
    pltpu.sync_copy(shared_cnt, allcnt_v)
    c_vec = _zi(0)
    w_vec = _zi(0)
    for sp in range(NS):
        row = allcnt_v[pl.ds(sp * L, L)]
        c_vec = c_vec + row
        before = _zi(sp) < _zi(s)
        w_vec = w_vec + jnp.where(before, row, _zi(0))
    nblk_vec = (c_vec + (B - 1)) // B
    excl = plsc.cumsum(nblk_vec) - nblk_vec
    base_vec = _zi(c * SREG) + B * excl + w_vec

    @pl.when(s == 0)
    def _():
        cntv_v[...] = c_vec
        pltpu.sync_copy(cntv_v, cnt.at[pl.ds(c * L, L)])

    cur_vec = base_vec
    for g in range(NG):
        for ids_v, out_v in ((id0_v, s0_v), (id1_v, s1_v)):
            ids_ = ids_v[pl.ds(g * L, L)]
            slk = _zi(0)
            for e in range(E):
                m = ids_ == e
                mi = m.astype(jnp.int32)
                incl = plsc.cumsum(mi)
                splat = _dyn_gather(cur_vec, _zi(e))
                slk = jnp.where(m, splat + incl - 1, slk)
                cur_vec = cur_vec + jnp.where(iota16 == e, _zi(jnp.sum(mi)),
                                              _zi(0))
            out_v[pl.ds(g * L, L)] = slk

    pltpu.sync_copy(s0_v, s0.at[pl.ds(tbase, TPW)])
    pltpu.sync_copy(s1_v, s1.at[pl.ds(tbase, TPW)])
    pltpu.sync_copy(w0_v, w0.at[pl.ds(tbase, TPW)])
    pltpu.sync_copy(w1_v, w1.at[pl.ds(tbase, TPW)])

    pltpu.sync_copy(hs.at[pl.ds(tbase, TPW)], xbuf_v)
    pltpu.async_copy(xbuf_v, xs.at[s0_v], dsem).wait()
    pltpu.async_copy(xbuf_v, xs.at[s1_v], dsem).wait()


_routing_call = functools.partial(
    pl.kernel,
    out_type=(
        jax.ShapeDtypeStruct((NR, H), jnp.float32),
        jax.ShapeDtypeStruct((T,), jnp.int32),
        jax.ShapeDtypeStruct((T,), jnp.int32),
        jax.ShapeDtypeStruct((T,), jnp.float32),
        jax.ShapeDtypeStruct((T,), jnp.float32),
        jax.ShapeDtypeStruct((NC * L,), jnp.int32),
    ),
    mesh=_mesh,
    compiler_params=pltpu.CompilerParams(needs_layout_passes=False),
    scratch_types=[
        pltpu.VMEM((TPW * E,), jnp.float32),
        pltpu.VMEM((TPW,), jnp.int32),
        pltpu.VMEM((TPW,), jnp.int32),
        pltpu.VMEM((TPW,), jnp.int32),
        pltpu.VMEM((TPW,), jnp.int32),
        pltpu.VMEM((TPW,), jnp.float32),
        pltpu.VMEM((TPW,), jnp.float32),
        pltpu.VMEM((L,), jnp.int32),
        pltpu.VMEM((NS * L,), jnp.int32),
        pltpu.VMEM((TPW, H), jnp.float32),
        pltpu.VMEM_SHARED((NS * L,), jnp.int32),
        pltpu.SemaphoreType.DMA,
    ],
)(_routing_body)


def _blockmap_body(cnt, bperm, eperm, cboth_v, bp_v, ep_v):
    c = lax.axis_index("c")
    s = lax.axis_index("s")

    @pl.when((c == 0) & (s == 0))
    def _():
        iota16 = lax.iota(jnp.int32, L)
        pltpu.sync_copy(cnt, cboth_v)
        cnt0 = cboth_v[pl.ds(0, L)]
        cnt1 = cboth_v[pl.ds(L, L)]
        nblk0 = (cnt0 + (B - 1)) // B
        nblk1 = (cnt1 + (B - 1)) // B
        st0 = plsc.cumsum(nblk0) - nblk0
        st1 = NBLK_C + plsc.cumsum(nblk1) - nblk1
        jh = iota16 // 2
        even = (iota16 % 2) == 0
        nb_em = jnp.where(even, _dyn_gather(nblk0, jh),
                          _dyn_gather(nblk1, jh))
        st_em = jnp.where(even, _dyn_gather(st0, jh),
                          _dyn_gather(st1, jh))
        bp_v[pl.ds(0, L)] = st_em
        ep_v[pl.ds(0, L)] = nb_em
        pltpu.sync_copy(bp_v, bperm)
        pltpu.sync_copy(ep_v, eperm)


_blockmap_call = functools.partial(
    pl.kernel,
    out_type=(
        jax.ShapeDtypeStruct((L,), jnp.int32),
        jax.ShapeDtypeStruct((L,), jnp.int32),
    ),
    mesh=_mesh,
    compiler_params=pltpu.CompilerParams(needs_layout_passes=False),
    scratch_types=[
        pltpu.VMEM((NC * L,), jnp.int32),
        pltpu.VMEM((L,), jnp.int32),
        pltpu.VMEM((L,), jnp.int32),
    ],
)(_blockmap_body)


def _mm_body(st, nb, xs, w13_ref, w2_ref, ys, xv, yv, actv, semx, semy):
    e = pl.program_id(0)
    n0 = nb[2 * e]
    n1 = nb[2 * e + 1]
    s0b = st[2 * e]
    s1b = st[2 * e + 1]
    n = n0 + n1

    def rowblk(j):
        return jnp.where(j < n0, s0b + j, s1b + (j - n0)) * B

    def start_x(j):
        sl = lax.rem(j, 2)
        pltpu.make_async_copy(xs.at[pl.ds(rowblk(j), B)], xv.at[sl],
                              semx).start()

    def emit_y(j):
        sl = lax.rem(j, 2)
        yv[sl] = lax.dot_general(actv[sl], w2_ref[0], (((1,), (1,)), ((), ())),
                                 preferred_element_type=jnp.float32)
        cp = pltpu.make_async_copy(yv.at[sl], ys.at[pl.ds(rowblk(j), B)], semy)
        cp.start()
        cp.wait()

    @pl.when(n > 0)
    def _():
        start_x(0)

    def step(j, carry):
        sl = lax.rem(j, 2)
        pltpu.make_async_copy(xs.at[pl.ds(rowblk(j), B)], xv.at[sl],
                              semx).wait()

        @pl.when(j + 1 < n)
        def _():
            start_x(j + 1)

        x = xv[sl]
        w1 = w13_ref[0, 0]
        w3 = w13_ref[0, 1]
        h1 = lax.dot_general(x, w1, (((1,), (1,)), ((), ())),
                             preferred_element_type=jnp.float32)
        h3 = lax.dot_general(x, w3, (((1,), (1,)), ((), ())),
                             preferred_element_type=jnp.float32)
        actv[sl] = h1 * jax.nn.sigmoid(h1) * h3

        @pl.when(j >= 1)
        def _():
            emit_y(j - 1)

        return carry

    lax.fori_loop(0, n, step, 0)

    @pl.when(n > 0)
    def _():
        emit_y(n - 1)


def _matmul_call(st, nb, xs, w13r, w2):
    return pl.pallas_call(
        _mm_body,
        grid_spec=pltpu.PrefetchScalarGridSpec(
            num_scalar_prefetch=2,
            grid=(E,),
            in_specs=[
                pl.BlockSpec(memory_space=pltpu.MemorySpace.HBM),
                pl.BlockSpec((1, 2, IDIM, H), lambda e, st, nb: (e, 0, 0, 0)),
                pl.BlockSpec((1, H, IDIM), lambda e, st, nb: (e, 0, 0)),
            ],
            out_specs=pl.BlockSpec(memory_space=pltpu.MemorySpace.HBM),
            scratch_shapes=[
                pltpu.VMEM((2, B, H), jnp.float32),
                pltpu.VMEM((2, B, H), jnp.float32),
                pltpu.VMEM((2, B, IDIM), jnp.float32),
                pltpu.SemaphoreType.DMA,
                pltpu.SemaphoreType.DMA,
            ],
        ),
        out_shape=jax.ShapeDtypeStruct((NR, H), jnp.float32),
        compiler_params=pltpu.CompilerParams(
            dimension_semantics=("arbitrary",),
        ),
    )(st, nb, xs, w13r, w2)


_CH = 32


def _combine_body(ys, s0, s1, w0, w1, out,
                  s0_v, s1_v, w0_v, w1_v, buf0, buf1, dsem):
    c = lax.axis_index("c")
    s = lax.axis_index("s")
    tbase = c * (T // NC) + s * TPW
    for ch in range(TPW // _CH):
        pltpu.sync_copy(s0.at[pl.ds(tbase + ch * _CH, _CH)], s0_v)
        pltpu.sync_copy(s1.at[pl.ds(tbase + ch * _CH, _CH)], s1_v)
        pltpu.sync_copy(w0.at[pl.ds(tbase + ch * _CH, _CH)], w0_v)
        pltpu.sync_copy(w1.at[pl.ds(tbase + ch * _CH, _CH)], w1_v)
        pltpu.async_copy(ys.at[s0_v], buf0, dsem).wait()
        pltpu.async_copy(ys.at[s1_v], buf1, dsem).wait()

        def row(r, carry):
            ws0 = plsc.load_gather(w0_v, [_zi(r)])
            ws1 = plsc.load_gather(w1_v, [_zi(r)])
            for v in range(H // L):
                a = buf0[r, pl.ds(v * L, L)]
                b = buf1[r, pl.ds(v * L, L)]
                buf0[r, pl.ds(v * L, L)] = ws0 * a + ws1 * b
            return carry

        lax.fori_loop(0, _CH, row, 0)
        pltpu.sync_copy(buf0, out.at[pl.ds(tbase + ch * _CH, _CH)])


_combine_call = functools.partial(
    pl.kernel,
    out_type=jax.ShapeDtypeStruct((T, H), jnp.float32),
    mesh=_mesh,
    compiler_params=pltpu.CompilerParams(needs_layout_passes=False),
    scratch_types=[
        pltpu.VMEM((_CH,), jnp.int32),
        pltpu.VMEM((_CH,), jnp.int32),
        pltpu.VMEM((_CH,), jnp.float32),
        pltpu.VMEM((_CH,), jnp.float32),
        pltpu.VMEM((_CH, H), jnp.float32),
        pltpu.VMEM((_CH, H), jnp.float32),
        pltpu.SemaphoreType.DMA,
    ],
)(_combine_body)


def kernel(hidden_states, router_logits, w13_weight, w2_weight):
    lg_flat = router_logits.reshape(T * E)
    w13r = w13_weight.reshape(E, 2, IDIM, H)
    xs, s0, s1, w0, w1, cnt = _routing_call(hidden_states, lg_flat)
    st, nb = _blockmap_call(cnt)
    ys = _matmul_call(st, nb, xs, w13r, w2_weight)
    return _combine_call(ys, s0, s1, w0, w1)


# speedup vs baseline: 1.2056x; 1.2056x over previous
import functools

import jax
import jax.numpy as jnp
from jax import lax
from jax.experimental import pallas as pl
from jax.experimental.pallas import tpu as pltpu
from jax.experimental.pallas import tpu_sc as plsc

T, H, IDIM, E, TOPK = 2048, 1024, 2048, 8, 2

NC, NS, L = 2, 16, 16
NW = NC * NS
TPW = T // NW
NG = TPW // L

B = 128
PAIRS_C = T * TOPK // NC
NBLK_C = PAIRS_C // B + E
SREG = NBLK_C * B
NR = NC * SREG
NBLK = NC * NBLK_C

_mesh = plsc.VectorSubcoreMesh(core_axis_name="c", subcore_axis_name="s")


def _zi(v):
    return jnp.zeros((L,), jnp.int32) + v


def _zf(v):
    return jnp.zeros((L,), jnp.float32) + v


_GDN = lax.GatherDimensionNumbers(offset_dims=(), collapsed_slice_dims=(0,),
                                  start_index_map=(0,))


def _dyn_gather(vec, idx):
    return lax.gather(vec, idx[:, None], _GDN, (1,),
                      mode=lax.GatherScatterMode.PROMISE_IN_BOUNDS)


def _routing_body(hs, lg, xs, s0, s1, w0, w1, cnt,
                  lg_v, id0_v, id1_v, s0_v, s1_v, w0_v, w1_v,
                  cntv_v, allcnt_v, xbuf_v, shared_cnt, dsem):
    c = lax.axis_index("c")
    s = lax.axis_index("s")
    tbase = c * (T // NC) + s * TPW
    iota16 = lax.iota(jnp.int32, L)

    pltpu.sync_copy(lg.at[pl.ds(tbase * E, TPW * E)], lg_v)
    cnt_sc = [jnp.int32(0)] * E
    for g in range(NG):
        lvec = [plsc.load_gather(lg_v, [iota16 * E + (g * L * E + e)])
                for e in range(E)]
        a1 = lvec[0]
        i1 = _zi(0)
        a2 = _zf(-jnp.inf)
        i2 = _zi(0)
        for e in range(1, E):
            le = lvec[e]
            gt1 = le > a1
            gt2 = le > a2
            ef = _zi(e)
            i2 = jnp.where(gt1, i1, jnp.where(gt2, ef, i2))
            a2 = jnp.where(gt1, a1, jnp.where(gt2, le, a2))
            i1 = jnp.where(gt1, ef, i1)
            a1 = jnp.where(gt1, le, a1)
        p2 = jnp.exp(a2 - a1)
        wt1 = 1.0 / (1.0 + p2)
        id0_v[pl.ds(g * L, L)] = i1
        id1_v[pl.ds(g * L, L)] = i2
        w0_v[pl.ds(g * L, L)] = wt1
        w1_v[pl.ds(g * L, L)] = p2 * wt1
        for e in range(E):
            m = (i1 == e).astype(jnp.int32) + (i2 == e).astype(jnp.int32)
            cnt_sc[e] = cnt_sc[e] + jnp.sum(m)

    cnt_vec = _zi(0)
    for e in range(E):
        cnt_vec = jnp.where(iota16 == e, _zi(cnt_sc[e]), cnt_vec)
    cntv_v[...] = cnt_vec
    pltpu.sync_copy(cntv_v, shared_cnt.at[pl.ds(s * L, L)])
    plsc.subcore_barrier()
    pltpu.sync_copy(shared_cnt, allcnt_v)
    c_vec = _zi(0)
    w_vec = _zi(0)
    for sp in range(NS):
        row = allcnt_v[pl.ds(sp * L, L)]
        c_vec = c_vec + row
        before = _zi(sp) < _zi(s)
        w_vec = w_vec + jnp.where(before, row, _zi(0))
    nblk_vec = (c_vec + (B - 1)) // B
    excl = plsc.cumsum(nblk_vec) - nblk_vec
    base_vec = _zi(c * SREG) + B * excl + w_vec

    @pl.when(s == 0)
    def _():
        cntv_v[...] = c_vec
        pltpu.sync_copy(cntv_v, cnt.at[pl.ds(c * L, L)])

    cur_vec = base_vec
    for g in range(NG):
        for ids_v, out_v in ((id0_v, s0_v), (id1_v, s1_v)):
            ids_ = ids_v[pl.ds(g * L, L)]
            slk = _zi(0)
            for e in range(E):
                m = ids_ == e
                mi = m.astype(jnp.int32)
                incl = plsc.cumsum(mi)
                splat = _dyn_gather(cur_vec, _zi(e))
                slk = jnp.where(m, splat + incl - 1, slk)
                cur_vec = cur_vec + jnp.where(iota16 == e, _zi(jnp.sum(mi)),
                                              _zi(0))
            out_v[pl.ds(g * L, L)] = slk

    pltpu.sync_copy(s0_v, s0.at[pl.ds(tbase, TPW)])
    pltpu.sync_copy(s1_v, s1.at[pl.ds(tbase, TPW)])
    pltpu.sync_copy(w0_v, w0.at[pl.ds(tbase, TPW)])
    pltpu.sync_copy(w1_v, w1.at[pl.ds(tbase, TPW)])

    pltpu.sync_copy(hs.at[pl.ds(tbase, TPW)], xbuf_v)
    pltpu.async_copy(xbuf_v, xs.at[s0_v], dsem).wait()
    pltpu.async_copy(xbuf_v, xs.at[s1_v], dsem).wait()


_routing_call = functools.partial(
    pl.kernel,
    out_type=(
        jax.ShapeDtypeStruct((NR, H), jnp.float32),
        jax.ShapeDtypeStruct((T,), jnp.int32),
        jax.ShapeDtypeStruct((T,), jnp.int32),
        jax.ShapeDtypeStruct((T,), jnp.float32),
        jax.ShapeDtypeStruct((T,), jnp.float32),
        jax.ShapeDtypeStruct((NC * L,), jnp.int32),
    ),
    mesh=_mesh,
    compiler_params=pltpu.CompilerParams(needs_layout_passes=False),
    scratch_types=[
        pltpu.VMEM((TPW * E,), jnp.float32),
        pltpu.VMEM((TPW,), jnp.int32),
        pltpu.VMEM((TPW,), jnp.int32),
        pltpu.VMEM((TPW,), jnp.int32),
        pltpu.VMEM((TPW,), jnp.int32),
        pltpu.VMEM((TPW,), jnp.float32),
        pltpu.VMEM((TPW,), jnp.float32),
        pltpu.VMEM((L,), jnp.int32),
        pltpu.VMEM((NS * L,), jnp.int32),
        pltpu.VMEM((TPW, H), jnp.float32),
        pltpu.VMEM_SHARED((NS * L,), jnp.int32),
        pltpu.SemaphoreType.DMA,
    ],
)(_routing_body)


def _blockmap_body(cnt, bperm, eperm, vperm, cboth_v, bp_v, ep_v, vp_v):
    c = lax.axis_index("c")
    s = lax.axis_index("s")

    @pl.when((c == 0) & (s == 0))
    def _():
        iota16 = lax.iota(jnp.int32, L)
        pltpu.sync_copy(cnt, cboth_v)
        cnt0 = cboth_v[pl.ds(0, L)]
        cnt1 = cboth_v[pl.ds(L, L)]
        nblk0 = (cnt0 + (B - 1)) // B
        nblk1 = (cnt1 + (B - 1)) // B
        st0 = plsc.cumsum(nblk0) - nblk0
        st1 = NBLK_C + plsc.cumsum(nblk1) - nblk1
        jh = iota16 // 2
        even = (iota16 % 2) == 0
        nb_em = jnp.where(even, _dyn_gather(nblk0, jh),
                          _dyn_gather(nblk1, jh))
        st_em = jnp.where(even, _dyn_gather(st0, jh),
                          _dyn_gather(st1, jh))
        cum_em = plsc.cumsum(nb_em) - nb_em
        nb_sc, st_sc, cum_sc = [], [], []
        lastb = jnp.int32(0)
        laste = jnp.int32(0)
        for j in range(2 * E):
            mj = iota16 == j
            nb_j = jnp.sum(jnp.where(mj, nb_em, _zi(0)))
            st_j = jnp.sum(jnp.where(mj, st_em, _zi(0)))
            cm_j = jnp.sum(jnp.where(mj, cum_em, _zi(0)))
            nb_sc.append(nb_j)
            st_sc.append(st_j)
            cum_sc.append(cm_j)
            lastb = jnp.where(nb_j > 0, st_j + nb_j - 1, lastb)
            laste = jnp.where(nb_j > 0, jnp.int32(j // 2), laste)
        for i in range(NBLK // L):
            gv = iota16 + L * i
            bp = _zi(lastb)
            ep = _zi(laste)
            vp = _zi(0)
            for j in range(2 * E):
                lo = _zi(cum_sc[j])
                m = (gv >= lo) & (gv < lo + _zi(nb_sc[j]))
                bp = jnp.where(m, _zi(st_sc[j]) + gv - lo, bp)
                ep = jnp.where(m, _zi(j // 2), ep)
                vp = jnp.where(m, _zi(1), vp)
            bp_v[pl.ds(L * i, L)] = bp
            ep_v[pl.ds(L * i, L)] = ep
            vp_v[pl.ds(L * i, L)] = vp
        pltpu.sync_copy(bp_v, bperm)
        pltpu.sync_copy(ep_v, eperm)
        pltpu.sync_copy(vp_v, vperm)


_blockmap_call = functools.partial(
    pl.kernel,
    out_type=(
        jax.ShapeDtypeStruct((NBLK,), jnp.int32),
        jax.ShapeDtypeStruct((NBLK,), jnp.int32),
        jax.ShapeDtypeStruct((NBLK,), jnp.int32),
    ),
    mesh=_mesh,
    compiler_params=pltpu.CompilerParams(needs_layout_passes=False),
    scratch_types=[
        pltpu.VMEM((NC * L,), jnp.int32),
        pltpu.VMEM((NBLK,), jnp.int32),
        pltpu.VMEM((NBLK,), jnp.int32),
        pltpu.VMEM((NBLK,), jnp.int32),
    ],
)(_blockmap_body)


def _mm_body(bp, ep, vp, x_ref, w13_ref, w2_ref, y_ref):
    g = pl.program_id(0)

    @pl.when(vp[g] == 1)
    def _():
        x = x_ref[...]
        w1 = w13_ref[0, 0]
        w3 = w13_ref[0, 1]
        h1 = lax.dot_general(x, w1, (((1,), (1,)), ((), ())),
                             preferred_element_type=jnp.float32)
        h3 = lax.dot_general(x, w3, (((1,), (1,)), ((), ())),
                             preferred_element_type=jnp.float32)
        act = h1 * jax.nn.sigmoid(h1) * h3
        y_ref[...] = lax.dot_general(act, w2_ref[0], (((1,), (1,)), ((), ())),
                                     preferred_element_type=jnp.float32)


def _matmul_call(bperm, eperm, vperm, xs, w13r, w2):
    return pl.pallas_call(
        _mm_body,
        grid_spec=pltpu.PrefetchScalarGridSpec(
            num_scalar_prefetch=3,
            grid=(NBLK,),
            in_specs=[
                pl.BlockSpec((B, H), lambda g, bp, ep, vp: (bp[g], 0)),
                pl.BlockSpec((1, 2, IDIM, H),
                             lambda g, bp, ep, vp: (ep[g], 0, 0, 0)),
                pl.BlockSpec((1, H, IDIM), lambda g, bp, ep, vp: (ep[g], 0, 0)),
            ],
            out_specs=pl.BlockSpec((B, H), lambda g, bp, ep, vp: (bp[g], 0)),
        ),
        out_shape=jax.ShapeDtypeStruct((NR, H), jnp.float32),
        compiler_params=pltpu.CompilerParams(
            dimension_semantics=("arbitrary",),
        ),
    )(bperm, eperm, vperm, xs, w13r, w2)


_CH = 32


def _combine_body(ys, s0, s1, w0, w1, out,
                  s0_v, s1_v, w0_v, w1_v, buf0, buf1, dsem):
    c = lax.axis_index("c")
    s = lax.axis_index("s")
    tbase = c * (T // NC) + s * TPW
    for ch in range(TPW // _CH):
        pltpu.sync_copy(s0.at[pl.ds(tbase + ch * _CH, _CH)], s0_v)
        pltpu.sync_copy(s1.at[pl.ds(tbase + ch * _CH, _CH)], s1_v)
        pltpu.sync_copy(w0.at[pl.ds(tbase + ch * _CH, _CH)], w0_v)
        pltpu.sync_copy(w1.at[pl.ds(tbase + ch * _CH, _CH)], w1_v)
        pltpu.async_copy(ys.at[s0_v], buf0, dsem).wait()
        pltpu.async_copy(ys.at[s1_v], buf1, dsem).wait()

        def row(r, carry):
            ws0 = plsc.load_gather(w0_v, [_zi(r)])
            ws1 = plsc.load_gather(w1_v, [_zi(r)])
            for v in range(H // L):
                a = buf0[r, pl.ds(v * L, L)]
                b = buf1[r, pl.ds(v * L, L)]
                buf0[r, pl.ds(v * L, L)] = ws0 * a + ws1 * b
            return carry

        lax.fori_loop(0, _CH, row, 0)
        pltpu.sync_copy(buf0, out.at[pl.ds(tbase + ch * _CH, _CH)])


_combine_call = functools.partial(
    pl.kernel,
    out_type=jax.ShapeDtypeStruct((T, H), jnp.float32),
    mesh=_mesh,
    compiler_params=pltpu.CompilerParams(needs_layout_passes=False),
    scratch_types=[
        pltpu.VMEM((_CH,), jnp.int32),
        pltpu.VMEM((_CH,), jnp.int32),
        pltpu.VMEM((_CH,), jnp.float32),
        pltpu.VMEM((_CH,), jnp.float32),
        pltpu.VMEM((_CH, H), jnp.float32),
        pltpu.VMEM((_CH, H), jnp.float32),
        pltpu.SemaphoreType.DMA,
    ],
)(_combine_body)


def kernel(hidden_states, router_logits, w13_weight, w2_weight):
    lg_flat = router_logits.reshape(T * E)
    w13r = w13_weight.reshape(E, 2, IDIM, H)
    xs, s0, s1, w0, w1, cnt = _routing_call(hidden_states, lg_flat)
    bperm, eperm, vperm = _blockmap_call(cnt)
    ys = _matmul_call(bperm, eperm, vperm, xs, w13r, w2_weight)
    return _combine_call(ys, s0, s1, w0, w1)


# speedup vs baseline: 1.2185x; 1.0107x over previous
import functools

import jax
import jax.numpy as jnp
from jax import lax
from jax.experimental import pallas as pl
from jax.experimental.pallas import tpu as pltpu
from jax.experimental.pallas import tpu_sc as plsc

T, H, IDIM, E, TOPK = 2048, 1024, 2048, 8, 2

NC, NS, L = 2, 16, 16
NW = NC * NS
TPW = T // NW
NG = TPW // L

B = 128
PAIRS_C = T * TOPK // NC
NBLK_C = PAIRS_C // B + E
SREG = NBLK_C * B
NR = NC * SREG
NBLK = NC * NBLK_C

_mesh = plsc.VectorSubcoreMesh(core_axis_name="c", subcore_axis_name="s")


def _zi(v):
    return jnp.zeros((L,), jnp.int32) + v


def _zf(v):
    return jnp.zeros((L,), jnp.float32) + v


_GDN = lax.GatherDimensionNumbers(offset_dims=(), collapsed_slice_dims=(0,),
                                  start_index_map=(0,))


def _dyn_gather(vec, idx):
    return lax.gather(vec, idx[:, None], _GDN, (1,),
                      mode=lax.GatherScatterMode.PROMISE_IN_BOUNDS)


def _routing_body(hs, lg, xs, s0, s1, w0, w1, cnt,
                  lg_v, id0_v, id1_v, s0_v, s1_v, w0_v, w1_v,
                  cntv_v, allcnt_v, xbuf_v, shared_cnt, dsem):
    c = lax.axis_index("c")
    s = lax.axis_index("s")
    tbase = c * (T // NC) + s * TPW
    iota16 = lax.iota(jnp.int32, L)

    pltpu.sync_copy(lg.at[pl.ds(tbase * E, TPW * E)], lg_v)
    cnt_sc = [jnp.int32(0)] * E
    for g in range(NG):
        lvec = [plsc.load_gather(lg_v, [iota16 * E + (g * L * E + e)])
                for e in range(E)]
        a1 = lvec[0]
        i1 = _zi(0)
        a2 = _zf(-jnp.inf)
        i2 = _zi(0)
        for e in range(1, E):
            le = lvec[e]
            gt1 = le > a1
            gt2 = le > a2
            ef = _zi(e)
            i2 = jnp.where(gt1, i1, jnp.where(gt2, ef, i2))
            a2 = jnp.where(gt1, a1, jnp.where(gt2, le, a2))
            i1 = jnp.where(gt1, ef, i1)
            a1 = jnp.where(gt1, le, a1)
        p2 = jnp.exp(a2 - a1)
        wt1 = 1.0 / (1.0 + p2)
        id0_v[pl.ds(g * L, L)] = i1
        id1_v[pl.ds(g * L, L)] = i2
        w0_v[pl.ds(g * L, L)] = wt1
        w1_v[pl.ds(g * L, L)] = p2 * wt1
        for e in range(E):
            m = (i1 == e).astype(jnp.int32) + (i2 == e).astype(jnp.int32)
            cnt_sc[e] = cnt_sc[e] + jnp.sum(m)

    cnt_vec = _zi(0)
    for e in range(E):
        cnt_vec = jnp.where(iota16 == e, _zi(cnt_sc[e]), cnt_vec)
    cntv_v[...] = cnt_vec
    pltpu.sync_copy(cntv_v, shared_cnt.at[pl.ds(s * L, L)])
    plsc.subcore_barrier()
    pltpu.sync_copy(shared_cnt, allcnt_v)
    c_vec = _zi(0)
    w_vec = _zi(0)
    for sp in range(NS):
        row = allcnt_v[pl.ds(sp * L, L)]
        c_vec = c_vec + row
        before = _zi(sp) < _zi(s)
        w_vec = w_vec + jnp.where(before, row, _zi(0))
    nblk_vec = (c_vec + (B - 1)) // B
    excl = plsc.cumsum(nblk_vec) - nblk_vec
    base_vec = _zi(c * SREG) + B * excl + w_vec

    @pl.when(s == 0)
    def _():
        cntv_v[...] = c_vec
        pltpu.sync_copy(cntv_v, cnt.at[pl.ds(c * L, L)])

    cur_vec = base_vec
    for g in range(NG):
        for ids_v, out_v in ((id0_v, s0_v), (id1_v, s1_v)):
            ids_ = ids_v[pl.ds(g * L, L)]
            slk = _zi(0)
            for e in range(E):
                m = ids_ == e
                mi = m.astype(jnp.int32)
                incl = plsc.cumsum(mi)
                splat = _dyn_gather(cur_vec, _zi(e))
                slk = jnp.where(m, splat + incl - 1, slk)
                cur_vec = cur_vec + jnp.where(iota16 == e, _zi(jnp.sum(mi)),
                                              _zi(0))
            out_v[pl.ds(g * L, L)] = slk

    pltpu.sync_copy(s0_v, s0.at[pl.ds(tbase, TPW)])
    pltpu.sync_copy(s1_v, s1.at[pl.ds(tbase, TPW)])
    pltpu.sync_copy(w0_v, w0.at[pl.ds(tbase, TPW)])
    pltpu.sync_copy(w1_v, w1.at[pl.ds(tbase, TPW)])

    pltpu.sync_copy(hs.at[pl.ds(tbase, TPW)], xbuf_v)
    cp0 = pltpu.async_copy(xbuf_v, xs.at[s0_v], dsem)
    cp1 = pltpu.async_copy(xbuf_v, xs.at[s1_v], dsem)
    cp0.wait()
    cp1.wait()


_routing_call = functools.partial(
    pl.kernel,
    out_type=(
        jax.ShapeDtypeStruct((NR, H), jnp.float32),
        jax.ShapeDtypeStruct((T,), jnp.int32),
        jax.ShapeDtypeStruct((T,), jnp.int32),
        jax.ShapeDtypeStruct((T,), jnp.float32),
        jax.ShapeDtypeStruct((T,), jnp.float32),
        jax.ShapeDtypeStruct((NC * L,), jnp.int32),
    ),
    mesh=_mesh,
    compiler_params=pltpu.CompilerParams(needs_layout_passes=False),
    scratch_types=[
        pltpu.VMEM((TPW * E,), jnp.float32),
        pltpu.VMEM((TPW,), jnp.int32),
        pltpu.VMEM((TPW,), jnp.int32),
        pltpu.VMEM((TPW,), jnp.int32),
        pltpu.VMEM((TPW,), jnp.int32),
        pltpu.VMEM((TPW,), jnp.float32),
        pltpu.VMEM((TPW,), jnp.float32),
        pltpu.VMEM((L,), jnp.int32),
        pltpu.VMEM((NS * L,), jnp.int32),
        pltpu.VMEM((TPW, H), jnp.float32),
        pltpu.VMEM_SHARED((NS * L,), jnp.int32),
        pltpu.SemaphoreType.DMA,
    ],
)(_routing_body)


def _blockmap_body(cnt, bperm, eperm, vperm, cboth_v, bp_v, ep_v, vp_v):
    c = lax.axis_index("c")
    s = lax.axis_index("s")

    @pl.when((c == 0) & (s == 0))
    def _():
        iota16 = lax.iota(jnp.int32, L)
        pltpu.sync_copy(cnt, cboth_v)
        cnt0 = cboth_v[pl.ds(0, L)]
        cnt1 = cboth_v[pl.ds(L, L)]
        nblk0 = (cnt0 + (B - 1)) // B
        nblk1 = (cnt1 + (B - 1)) // B
        st0 = plsc.cumsum(nblk0) - nblk0
        st1 = NBLK_C + plsc.cumsum(nblk1) - nblk1
        jh = iota16 // 2
        even = (iota16 % 2) == 0
        nb_em = jnp.where(even, _dyn_gather(nblk0, jh),
                          _dyn_gather(nblk1, jh))
        st_em = jnp.where(even, _dyn_gather(st0, jh),
                          _dyn_gather(st1, jh))
        cum_em = plsc.cumsum(nb_em) - nb_em
        nb_sc, st_sc, cum_sc = [], [], []
        lastb = jnp.int32(0)
        laste = jnp.int32(0)
        for j in range(2 * E):
            mj = iota16 == j
            nb_j = jnp.sum(jnp.where(mj, nb_em, _zi(0)))
            st_j = jnp.sum(jnp.where(mj, st_em, _zi(0)))
            cm_j = jnp.sum(jnp.where(mj, cum_em, _zi(0)))
            nb_sc.append(nb_j)
            st_sc.append(st_j)
            cum_sc.append(cm_j)
            lastb = jnp.where(nb_j > 0, st_j + nb_j - 1, lastb)
            laste = jnp.where(nb_j > 0, jnp.int32(j // 2), laste)
        for i in range(NBLK // L):
            gv = iota16 + L * i
            bp = _zi(lastb)
            ep = _zi(laste)
            vp = _zi(0)
            for j in range(2 * E):
                lo = _zi(cum_sc[j])
                m = (gv >= lo) & (gv < lo + _zi(nb_sc[j]))
                bp = jnp.where(m, _zi(st_sc[j]) + gv - lo, bp)
                ep = jnp.where(m, _zi(j // 2), ep)
                vp = jnp.where(m, _zi(1), vp)
            bp_v[pl.ds(L * i, L)] = bp
            ep_v[pl.ds(L * i, L)] = ep
            vp_v[pl.ds(L * i, L)] = vp
        pltpu.sync_copy(bp_v, bperm)
        pltpu.sync_copy(ep_v, eperm)
        pltpu.sync_copy(vp_v, vperm)


_blockmap_call = functools.partial(
    pl.kernel,
    out_type=(
        jax.ShapeDtypeStruct((NBLK,), jnp.int32),
        jax.ShapeDtypeStruct((NBLK,), jnp.int32),
        jax.ShapeDtypeStruct((NBLK,), jnp.int32),
    ),
    mesh=_mesh,
    compiler_params=pltpu.CompilerParams(needs_layout_passes=False),
    scratch_types=[
        pltpu.VMEM((NC * L,), jnp.int32),
        pltpu.VMEM((NBLK,), jnp.int32),
        pltpu.VMEM((NBLK,), jnp.int32),
        pltpu.VMEM((NBLK,), jnp.int32),
    ],
)(_blockmap_body)


def _mm_body(bp, ep, vp, x_ref, w13_ref, w2_ref, y_ref):
    g = pl.program_id(0)

    @pl.when(vp[g] == 1)
    def _():
        x = x_ref[...]
        w1 = w13_ref[0, 0]
        w3 = w13_ref[0, 1]
        h1 = lax.dot_general(x, w1, (((1,), (1,)), ((), ())),
                             preferred_element_type=jnp.float32)
        h3 = lax.dot_general(x, w3, (((1,), (1,)), ((), ())),
                             preferred_element_type=jnp.float32)
        act = h1 * jax.nn.sigmoid(h1) * h3
        y_ref[...] = lax.dot_general(act, w2_ref[0], (((1,), (1,)), ((), ())),
                                     preferred_element_type=jnp.float32)


def _matmul_call(bperm, eperm, vperm, xs, w13r, w2):
    return pl.pallas_call(
        _mm_body,
        grid_spec=pltpu.PrefetchScalarGridSpec(
            num_scalar_prefetch=3,
            grid=(NBLK,),
            in_specs=[
                pl.BlockSpec((B, H), lambda g, bp, ep, vp: (bp[g], 0)),
                pl.BlockSpec((1, 2, IDIM, H),
                             lambda g, bp, ep, vp: (ep[g], 0, 0, 0)),
                pl.BlockSpec((1, H, IDIM), lambda g, bp, ep, vp: (ep[g], 0, 0)),
            ],
            out_specs=pl.BlockSpec((B, H), lambda g, bp, ep, vp: (bp[g], 0)),
        ),
        out_shape=jax.ShapeDtypeStruct((NR, H), jnp.float32),
        compiler_params=pltpu.CompilerParams(
            dimension_semantics=("arbitrary",),
        ),
    )(bperm, eperm, vperm, xs, w13r, w2)


_CH = 32


def _combine_body(ys, s0, s1, w0, w1, out,
                  s0_v, s1_v, w0_v, w1_v, buf0, buf1, dsem):
    c = lax.axis_index("c")
    s = lax.axis_index("s")
    tbase = c * (T // NC) + s * TPW
    for ch in range(TPW // _CH):
        pltpu.sync_copy(s0.at[pl.ds(tbase + ch * _CH, _CH)], s0_v)
        pltpu.sync_copy(s1.at[pl.ds(tbase + ch * _CH, _CH)], s1_v)
        pltpu.sync_copy(w0.at[pl.ds(tbase + ch * _CH, _CH)], w0_v)
        pltpu.sync_copy(w1.at[pl.ds(tbase + ch * _CH, _CH)], w1_v)
        cp0 = pltpu.async_copy(ys.at[s0_v], buf0, dsem)
        cp1 = pltpu.async_copy(ys.at[s1_v], buf1, dsem)
        cp0.wait()
        cp1.wait()

        def row(r, carry):
            ws0 = plsc.load_gather(w0_v, [_zi(r)])
            ws1 = plsc.load_gather(w1_v, [_zi(r)])
            for v in range(H // L):
                a = buf0[r, pl.ds(v * L, L)]
                b = buf1[r, pl.ds(v * L, L)]
                buf0[r, pl.ds(v * L, L)] = ws0 * a + ws1 * b
            return carry

        lax.fori_loop(0, _CH, row, 0)
        pltpu.sync_copy(buf0, out.at[pl.ds(tbase + ch * _CH, _CH)])


_combine_call = functools.partial(
    pl.kernel,
    out_type=jax.ShapeDtypeStruct((T, H), jnp.float32),
    mesh=_mesh,
    compiler_params=pltpu.CompilerParams(needs_layout_passes=False),
    scratch_types=[
        pltpu.VMEM((_CH,), jnp.int32),
        pltpu.VMEM((_CH,), jnp.int32),
        pltpu.VMEM((_CH,), jnp.float32),
        pltpu.VMEM((_CH,), jnp.float32),
        pltpu.VMEM((_CH, H), jnp.float32),
        pltpu.VMEM((_CH, H), jnp.float32),
        pltpu.SemaphoreType.DMA,
    ],
)(_combine_body)


def kernel(hidden_states, router_logits, w13_weight, w2_weight):
    lg_flat = router_logits.reshape(T * E)
    w13r = w13_weight.reshape(E, 2, IDIM, H)
    xs, s0, s1, w0, w1, cnt = _routing_call(hidden_states, lg_flat)
    bperm, eperm, vperm = _blockmap_call(cnt)
    ys = _matmul_call(bperm, eperm, vperm, xs, w13r, w2_weight)
    return _combine_call(ys, s0, s1, w0, w1)
